# Initial kernel scaffold; baseline (speedup 1.0000x reference)
import jax
import jax.numpy as jnp
from jax.experimental import pallas as pl


def kernel(indices, values, m, n, weight, bias):
    raise NotImplementedError("write your pallas kernel here")


# speedup vs baseline: 6.2971x; 6.2971x over previous
import functools

import jax
import jax.numpy as jnp
from jax import lax
from jax.experimental import pallas as pl
from jax.experimental.pallas import tpu as pltpu
from jax.experimental.pallas import tpu_sc as plsc

_M = 16384
_D = 64
_HALF = _D // 2
_NC = 2
_NS = 16
_B = 128
_ROWS_PER_TILE = _M // _NS


def _splat_idx(k):
    return jnp.full((16,), k, dtype=jnp.int32)


def _body(col_h, row_h, val_h, wlo_h, whi_h, bias_h, out_h,
          colv, rowv, valsv, rows_v, bias_v, obuf, acc, sem,
          *, num_batches):
    c = lax.axis_index("c")
    s = lax.axis_index("s")

    zeros16 = jnp.zeros((16,), jnp.float32)

    def zero_row(r, carry):
        obuf[r, pl.ds(0, 16)] = zeros16
        obuf[r, pl.ds(16, 16)] = zeros16
        return carry

    lax.fori_loop(0, _ROWS_PER_TILE, zero_row, 0)
    pltpu.sync_copy(obuf, acc.at[pl.ds(s * _ROWS_PER_TILE, _ROWS_PER_TILE)])
    pltpu.sync_copy(bias_h, bias_v)
    plsc.subcore_barrier()

    base0 = s * (num_batches * _B)

    def batch(i, carry):
        base = base0 + i * _B
        pltpu.sync_copy(col_h.at[pl.ds(base, _B)], colv)
        pltpu.sync_copy(row_h.at[pl.ds(base, _B)], rowv)
        pltpu.sync_copy(val_h.at[pl.ds(base, _B)], valsv)

        @pl.when(c == 0)
        def _():
            pltpu.async_copy(wlo_h.at[colv], rows_v, sem).wait()

        @pl.when(c == 1)
        def _():
            pltpu.async_copy(whi_h.at[colv], rows_v, sem).wait()

        def scale(j, carry2):
            vals16 = valsv[pl.ds(j * 16, 16)]
            for k in range(16):
                sp = vals16.at[_splat_idx(k)].get(mode="promise_in_bounds")
                b = j * 16 + k
                rows_v[b, pl.ds(0, 16)] = rows_v[b, pl.ds(0, 16)] * sp
                rows_v[b, pl.ds(16, 16)] = rows_v[b, pl.ds(16, 16)] * sp
            return carry2

        lax.fori_loop(0, _B // 16, scale, 0)
        pltpu.sync_copy(rows_v, acc.at[rowv], add=True)
        return carry

    lax.fori_loop(0, num_batches, batch, 0)
    plsc.subcore_barrier()

    pltpu.sync_copy(acc.at[pl.ds(s * _ROWS_PER_TILE, _ROWS_PER_TILE)], obuf)
    b_lo = bias_v[pl.ds(c * _HALF, 16)]
    b_hi = bias_v[pl.ds(c * _HALF + 16, 16)]

    def add_bias(r, carry):
        obuf[r, pl.ds(0, 16)] = obuf[r, pl.ds(0, 16)] + b_lo
        obuf[r, pl.ds(16, 16)] = obuf[r, pl.ds(16, 16)] + b_hi
        return carry

    lax.fori_loop(0, _ROWS_PER_TILE, add_bias, 0)
    pltpu.sync_copy(
        obuf,
        out_h.at[c, pl.ds(s * _ROWS_PER_TILE, _ROWS_PER_TILE)])


def kernel(indices, values, m, n, weight, bias):
    nnz = values.shape[0]
    chunk = _NS * _B
    num_batches = -(-nnz // chunk)
    nnz_pad = num_batches * chunk
    pad = nnz_pad - nnz
    col = jnp.concatenate([indices[1], jnp.zeros((pad,), jnp.int32)])
    row = jnp.concatenate([indices[0], jnp.zeros((pad,), jnp.int32)])
    val = jnp.concatenate([values, jnp.zeros((pad,), jnp.float32)])
    w_lo = weight[:, :_HALF]
    w_hi = weight[:, _HALF:]

    mesh = plsc.VectorSubcoreMesh(
        core_axis_name="c", subcore_axis_name="s",
        num_cores=_NC, num_subcores=_NS)
    f = pl.kernel(
        functools.partial(_body, num_batches=num_batches),
        out_type=jax.ShapeDtypeStruct((_NC, _M, _HALF), jnp.float32),
        mesh=mesh,
        compiler_params=pltpu.CompilerParams(use_tc_tiling_on_sc=False),
        scratch_types=[
            pltpu.VMEM((_B,), jnp.int32),
            pltpu.VMEM((_B,), jnp.int32),
            pltpu.VMEM((_B,), jnp.float32),
            pltpu.VMEM((_B, _HALF), jnp.float32),
            pltpu.VMEM((_D,), jnp.float32),
            pltpu.VMEM((_ROWS_PER_TILE, _HALF), jnp.float32),
            pltpu.VMEM_SHARED((_M, _HALF), jnp.float32),
            pltpu.SemaphoreType.DMA,
        ],
    )
    out = f(col, row, val, w_lo, w_hi, bias)
    return jnp.concatenate([out[0], out[1]], axis=1)


# speedup vs baseline: 16.1026x; 2.5571x over previous
import functools

import jax
import jax.numpy as jnp
from jax import lax
from jax.experimental import pallas as pl
from jax.experimental.pallas import tpu as pltpu
from jax.experimental.pallas import tpu_sc as plsc

_M = 16384
_D = 64
_HALF = _D // 2
_NC = 2
_NS = 16
_B = 128
_ROWS_PER_TILE = _M // _NS


def _splat_idx(k):
    return jnp.full((16,), k, dtype=jnp.int32)


def _scale(vv, rows):
    for j in range(_B // 16):
        vals16 = vv[pl.ds(j * 16, 16)]
        for k in range(16):
            sp = vals16.at[_splat_idx(k)].get(mode="promise_in_bounds")
            b = j * 16 + k
            rows[b, pl.ds(0, 16)] = rows[b, pl.ds(0, 16)] * sp
            rows[b, pl.ds(16, 16)] = rows[b, pl.ds(16, 16)] * sp


def _body(pk_h, vv_h, wlo_h, whi_h, bias_h, out_h,
          pk0, pk1, vv0, vv1, rows0, rows1, bias_v, obuf, acc,
          sem_pk0, sem_pk1, sem_vv0, sem_vv1, sem_g0, sem_g1,
          *, num_batches):
    c = lax.axis_index("c")
    s = lax.axis_index("s")
    pkv = (pk0, pk1)
    vv = (vv0, vv1)
    rows = (rows0, rows1)
    sem_pk = (sem_pk0, sem_pk1)
    sem_vv = (sem_vv0, sem_vv1)
    sem_g = (sem_g0, sem_g1)

    zeros16 = jnp.zeros((16,), jnp.float32)

    def zero_row(r, carry):
        obuf[r, pl.ds(0, 16)] = zeros16
        obuf[r, pl.ds(16, 16)] = zeros16
        return carry

    lax.fori_loop(0, _ROWS_PER_TILE, zero_row, 0)
    pltpu.sync_copy(obuf, acc.at[pl.ds(s * _ROWS_PER_TILE, _ROWS_PER_TILE)])
    pltpu.sync_copy(bias_h, bias_v)
    plsc.subcore_barrier()

    base0 = s * num_batches

    def issue_gather(q):
        colv = pkv[q].at[0]

        @pl.when(c == 0)
        def _():
            pltpu.async_copy(wlo_h.at[colv], rows[q], sem_g[q])

        @pl.when(c == 1)
        def _():
            pltpu.async_copy(whi_h.at[colv], rows[q], sem_g[q])

    def wait_pk(q):
        pltpu.make_async_copy(pk_h.at[0], pkv[q], sem_pk[q]).wait()

    def issue_pk(q, g):
        pltpu.async_copy(pk_h.at[g], pkv[q], sem_pk[q])
        pltpu.async_copy(vv_h.at[g], vv[q], sem_vv[q])

    def wait_vv(q):
        pltpu.make_async_copy(vv_h.at[0], vv[q], sem_vv[q]).wait()

    def wait_gather(q):
        pltpu.make_async_copy(wlo_h.at[pl.ds(0, _B)], rows[q], sem_g[q]).wait()

    issue_pk(0, base0)
    issue_pk(1, base0 + 1)
    wait_pk(0)
    issue_gather(0)

    def step(i2, carry):
        for p in (0, 1):
            i = i2 * 2 + p
            q = 1 - p
            wait_pk(q)
            issue_gather(q)
            wait_gather(p)
            wait_vv(p)
            _scale(vv[p], rows[p])
            pltpu.sync_copy(rows[p], acc.at[pkv[p].at[1]], add=True)
            nxt = jnp.minimum(i + 2, num_batches - 1)
            issue_pk(p, base0 + nxt)
        return carry

    lax.fori_loop(0, num_batches // 2, step, 0)
    wait_pk(1)
    wait_vv(1)
    wait_gather(0)
    plsc.subcore_barrier()

    pltpu.sync_copy(acc.at[pl.ds(s * _ROWS_PER_TILE, _ROWS_PER_TILE)], obuf)
    b_lo = bias_v[pl.ds(c * _HALF, 16)]
    b_hi = bias_v[pl.ds(c * _HALF + 16, 16)]

    def add_bias(r, carry):
        obuf[r, pl.ds(0, 16)] = obuf[r, pl.ds(0, 16)] + b_lo
        obuf[r, pl.ds(16, 16)] = obuf[r, pl.ds(16, 16)] + b_hi
        return carry

    lax.fori_loop(0, _ROWS_PER_TILE, add_bias, 0)
    pltpu.sync_copy(
        obuf,
        out_h.at[c, pl.ds(s * _ROWS_PER_TILE, _ROWS_PER_TILE)])


def kernel(indices, values, m, n, weight, bias):
    nnz = values.shape[0]
    chunk = _NS * _B * 2
    num_batches = 2 * (-(-nnz // chunk))
    nnz_pad = num_batches * _NS * _B
    pad = nnz_pad - nnz
    col = jnp.concatenate([indices[1], jnp.zeros((pad,), jnp.int32)])
    row = jnp.concatenate([indices[0], jnp.zeros((pad,), jnp.int32)])
    val = jnp.concatenate([values, jnp.zeros((pad,), jnp.float32)])
    tot = nnz_pad // _B
    pk = jnp.stack(
        [col.reshape(tot, _B), row.reshape(tot, _B)],
        axis=1)
    vv = val.reshape(tot, _B)
    w_lo = weight[:, :_HALF]
    w_hi = weight[:, _HALF:]

    mesh = plsc.VectorSubcoreMesh(
        core_axis_name="c", subcore_axis_name="s",
        num_cores=_NC, num_subcores=_NS)
    f = pl.kernel(
        functools.partial(_body, num_batches=num_batches),
        out_type=jax.ShapeDtypeStruct((_NC, _M, _HALF), jnp.float32),
        mesh=mesh,
        compiler_params=pltpu.CompilerParams(use_tc_tiling_on_sc=False),
        scratch_types=[
            pltpu.VMEM((2, _B), jnp.int32),
            pltpu.VMEM((2, _B), jnp.int32),
            pltpu.VMEM((_B,), jnp.float32),
            pltpu.VMEM((_B,), jnp.float32),
            pltpu.VMEM((_B, _HALF), jnp.float32),
            pltpu.VMEM((_B, _HALF), jnp.float32),
            pltpu.VMEM((_D,), jnp.float32),
            pltpu.VMEM((_ROWS_PER_TILE, _HALF), jnp.float32),
            pltpu.VMEM_SHARED((_M, _HALF), jnp.float32),
            pltpu.SemaphoreType.DMA,
            pltpu.SemaphoreType.DMA,
            pltpu.SemaphoreType.DMA,
            pltpu.SemaphoreType.DMA,
            pltpu.SemaphoreType.DMA,
            pltpu.SemaphoreType.DMA,
        ],
    )
    out = f(pk, vv, w_lo, w_hi, bias)
    return jnp.concatenate([out[0], out[1]], axis=1)
